# Initial kernel scaffold; baseline (speedup 1.0000x reference)
import jax
import jax.numpy as jnp
from jax.experimental import pallas as pl


def kernel(x, Wq, bq, Wk, bk, Wv, bv, Wo, bo):
    raise NotImplementedError("write your pallas kernel here")


# speedup vs baseline: 6.1458x; 6.1458x over previous
import functools
import math

import numpy as np
import jax
import jax.numpy as jnp
from jax.experimental import pallas as pl
from jax.experimental.pallas import tpu as pltpu

_L = 2048
_D = 768
_H = 12
_DH = 64
_U = 40
_UPAD = 64
_KC = 512
_CC = 256
_SCALE = 1.0 / math.sqrt(_DH)
_F32 = jnp.float32
_HIGH = jax.lax.Precision.HIGHEST


def _sample_counts_t():
    idx = np.asarray(jax.random.randint(jax.random.key(42), (_L, _U), 0, _L))
    c = np.zeros((_L, _L), np.float32)
    np.add.at(c, (np.arange(_L)[:, None], idx), 1.0)
    return np.ascontiguousarray(c.T)


_CT_NP = _sample_counts_t()


def _mm(a, b, ca, cb):
    return jax.lax.dot_general(
        a, b, (((ca,), (cb,)), ((), ())),
        precision=_HIGH, preferred_element_type=_F32)


def _mmb(a, b, ca, cb):
    return jax.lax.dot_general(
        a.astype(jnp.bfloat16), b.astype(jnp.bfloat16),
        (((ca,), (cb,)), ((), ())), preferred_element_type=_F32)


def _head_kernel(x_ref, ct_ref, wq_ref, bq_ref, wk_ref, bk_ref, wv_ref, bv_ref,
                 o_ref, p_ref):
    x = x_ref[...]
    q = _mmb(x, wq_ref[...], 1, 1) + bq_ref[0]
    k = _mmb(x, wk_ref[...], 1, 1) + bk_ref[0]
    v = _mmb(x, wv_ref[...], 1, 1) + bv_ref[0]

    mmax = jnp.full((1, _L), -jnp.inf, _F32)
    msum = jnp.zeros((1, _L), _F32)
    for c in range(_L // _KC):
        kc = k[c * _KC:(c + 1) * _KC, :]
        st = _mmb(kc, q, 1, 1)
        ctc = ct_ref[c * _KC:(c + 1) * _KC, :]
        mmax = jnp.maximum(
            mmax,
            jnp.max(jnp.where(ctc > 0.0, st, -jnp.inf), axis=0, keepdims=True))
        msum = msum + jnp.sum(st * ctc, axis=0, keepdims=True)
    m = mmax - msum * (1.0 / _L)

    iot = jax.lax.broadcasted_iota(jnp.int32, (1, _L), 1)
    p_ref[...] = jnp.zeros((_UPAD, _L), _F32)
    for i in range(_U):
        cur = jnp.max(m, axis=1, keepdims=True)
        pos = jnp.min(jnp.where(m == cur, iot, _L), axis=1, keepdims=True)
        hit = iot == pos
        p_ref[i:i + 1, :] = hit.astype(_F32)
        m = jnp.where(hit, -jnp.inf, m)
    p = p_ref[...]

    qs = _mm(p, q, 1, 0)
    iotf = iot.astype(_F32)
    qpos = jnp.sum(p * iotf, axis=1, keepdims=True)
    sc = _mmb(qs, k, 1, 1) * _SCALE
    sc = jnp.where(iotf > qpos, -jnp.inf, sc)
    smax = jnp.max(sc, axis=1, keepdims=True)
    e = jnp.exp(sc - smax)
    attn = e / jnp.sum(e, axis=1, keepdims=True)
    upd = _mmb(attn, v, 1, 0)

    rc = jax.lax.broadcasted_iota(jnp.int32, (_CC, _CC), 0)
    cc = jax.lax.broadcasted_iota(jnp.int32, (_CC, _CC), 1)
    tri = (cc <= rc).astype(_F32)
    carry = jnp.zeros((1, _DH), _F32)
    chunks = []
    for c in range(_L // _CC):
        vc = v[c * _CC:(c + 1) * _CC, :]
        chunks.append(_mm(tri, vc, 1, 0) + carry)
        carry = carry + jnp.sum(vc, axis=0, keepdims=True)
    ctx = jnp.concatenate(chunks, axis=0)

    selcol = _mm(p, jnp.ones((_UPAD, 1), _F32), 0, 0)
    scat = _mm(p, upd, 0, 0)
    o_ref[0] = ctx * (1.0 - selcol) + scat


def _proj_kernel(c_ref, wo_ref, bo_ref, o_ref):
    acc = bo_ref[...]
    for h in range(_H):
        acc = acc + _mmb(c_ref[h], wo_ref[:, h * _DH:(h + 1) * _DH], 1, 1)
    o_ref[...] = acc


def kernel(x, Wq, bq, Wk, bk, Wv, bv, Wo, bo):
    ct = jnp.asarray(_CT_NP)
    xs = x[0]
    bq2 = bq.reshape(_H, 1, _DH)
    bk2 = bk.reshape(_H, 1, _DH)
    bv2 = bv.reshape(_H, 1, _DH)
    bo2 = bo.reshape(1, _D)

    ctx = pl.pallas_call(
        _head_kernel,
        grid=(_H,),
        in_specs=[
            pl.BlockSpec((_L, _D), lambda h: (0, 0)),
            pl.BlockSpec((_L, _L), lambda h: (0, 0)),
            pl.BlockSpec((_DH, _D), lambda h: (h, 0)),
            pl.BlockSpec((1, 1, _DH), lambda h: (h, 0, 0)),
            pl.BlockSpec((_DH, _D), lambda h: (h, 0)),
            pl.BlockSpec((1, 1, _DH), lambda h: (h, 0, 0)),
            pl.BlockSpec((_DH, _D), lambda h: (h, 0)),
            pl.BlockSpec((1, 1, _DH), lambda h: (h, 0, 0)),
        ],
        out_specs=pl.BlockSpec((1, _L, _DH), lambda h: (h, 0, 0)),
        out_shape=jax.ShapeDtypeStruct((_H, _L, _DH), _F32),
        scratch_shapes=[pltpu.VMEM((_UPAD, _L), _F32)],
    )(xs, ct, Wq, bq2, Wk, bk2, Wv, bv2)

    out = pl.pallas_call(
        _proj_kernel,
        grid=(8,),
        in_specs=[
            pl.BlockSpec((_H, _L // 8, _DH), lambda i: (0, i, 0)),
            pl.BlockSpec((_D, _D), lambda i: (0, 0)),
            pl.BlockSpec((1, _D), lambda i: (0, 0)),
        ],
        out_specs=pl.BlockSpec((_L // 8, _D), lambda i: (i, 0)),
        out_shape=jax.ShapeDtypeStruct((_L, _D), _F32),
    )(ctx, Wo, bo2)
    return out[None]


# speedup vs baseline: 6.2986x; 1.0249x over previous
import math

import numpy as np
import jax
import jax.numpy as jnp
from jax.experimental import pallas as pl
from jax.experimental.pallas import tpu as pltpu

_L = 2048
_D = 768
_H = 12
_DH = 64
_U = 40
_UPAD = 64
_KC = 512
_CC = 256
_SCALE = 1.0 / math.sqrt(_DH)
_F32 = jnp.float32


def _rotl(x, r):
    return ((x << np.uint32(r)) | (x >> np.uint32(32 - r))).astype(np.uint32)


def _tf2x32_pair(k0, k1, x0, x1):
    ks = [np.uint32(k0), np.uint32(k1),
          np.uint32(np.uint32(k0) ^ np.uint32(k1) ^ np.uint32(0x1BD11BDA))]
    rot = [[13, 15, 26, 6], [17, 29, 16, 24]]
    x = [x0.astype(np.uint32) + ks[0], x1.astype(np.uint32) + ks[1]]
    for i in range(5):
        for r in rot[i % 2]:
            x[0] = x[0] + x[1]
            x[1] = _rotl(x[1], r)
            x[1] = x[1] ^ x[0]
        x[0] = x[0] + ks[(i + 1) % 3]
        x[1] = x[1] + ks[(i + 2) % 3] + np.uint32(i + 1)
    return x[0], x[1]


def _random_bits32(k0, k1, n):
    idx = np.arange(n, dtype=np.uint64)
    b1, b2 = _tf2x32_pair(k0, k1, (idx >> np.uint64(32)).astype(np.uint32),
                          (idx & np.uint64(0xFFFFFFFF)).astype(np.uint32))
    return b1 ^ b2


def _np_randint(seed, n, span):
    k0, k1 = np.uint32((seed >> 32) & 0xFFFFFFFF), np.uint32(seed & 0xFFFFFFFF)
    b1, b2 = _tf2x32_pair(k0, k1, np.zeros(2, np.uint32),
                          np.arange(2, dtype=np.uint32))
    hi = _random_bits32(b1[0], b2[0], n)
    lo = _random_bits32(b1[1], b2[1], n)
    span = np.uint32(span)
    mult = np.uint32((np.uint64(65536) % span) ** 2 % span)
    return (((hi % span) * mult + (lo % span)) % span).astype(np.int32)


def _sample_counts_t():
    idx = _np_randint(42, _L * _U, _L).reshape(_L, _U)
    c = np.zeros((_L, _L), np.float32)
    np.add.at(c, (np.arange(_L)[:, None], idx), 1.0)
    return np.ascontiguousarray(c.T)


_CT_NP = _sample_counts_t()
_BM_NP = np.where(_CT_NP > 0, np.float32(0), np.float32(-np.inf))


def _mmb(a, b, ca, cb):
    return jax.lax.dot_general(
        a.astype(jnp.bfloat16), b.astype(jnp.bfloat16),
        (((ca,), (cb,)), ((), ())), preferred_element_type=_F32)


def _head_kernel(x_ref, ct_ref, bm_ref, wq_ref, bq_ref, bqc_ref, wk_ref,
                 bk_ref, wv_ref, bv_ref, o_ref, p_ref):
    x = x_ref[...]
    q = _mmb(x, wq_ref[...], 1, 1) + bq_ref[0]
    k = _mmb(x, wk_ref[...], 1, 1) + bk_ref[0]
    v = _mmb(x, wv_ref[...], 1, 1) + bv_ref[0]
    qt = _mmb(wq_ref[...], x, 1, 1) + bqc_ref[0]

    kct = _mmb(k, ct_ref[...], 0, 0)
    msum = jnp.sum(qt.astype(jnp.bfloat16).astype(_F32) * kct,
                   axis=0, keepdims=True)
    mmax = jnp.full((1, _L), -jnp.inf, _F32)
    for c in range(_L // _KC):
        kc = k[c * _KC:(c + 1) * _KC, :]
        st = _mmb(kc, q, 1, 1)
        st = st + bm_ref[c * _KC:(c + 1) * _KC, :]
        mmax = jnp.maximum(mmax, jnp.max(st, axis=0, keepdims=True))
    m = mmax - msum * (1.0 / _L)

    iot = jax.lax.broadcasted_iota(jnp.int32, (1, _L), 1)
    p_ref[...] = jnp.zeros((_UPAD, _L), _F32)
    for i in range(_U):
        cur = jnp.max(m, axis=1, keepdims=True)
        pos = jnp.min(jnp.where(m == cur, iot, _L), axis=1, keepdims=True)
        hit = iot == pos
        p_ref[i:i + 1, :] = hit.astype(_F32)
        m = jnp.where(hit, -jnp.inf, m)
    p = p_ref[...]

    qs = _mmb(p, q, 1, 0)
    iotf = iot.astype(_F32)
    qpos = jnp.sum(p * iotf, axis=1, keepdims=True)
    sc = _mmb(qs, k, 1, 1) * _SCALE
    sc = jnp.where(iotf > qpos, -jnp.inf, sc)
    smax = jnp.max(sc, axis=1, keepdims=True)
    e = jnp.exp(sc - smax)
    attn = e / jnp.sum(e, axis=1, keepdims=True)
    upd = _mmb(attn, v, 1, 0)

    rc = jax.lax.broadcasted_iota(jnp.int32, (_CC, _CC), 0)
    cc = jax.lax.broadcasted_iota(jnp.int32, (_CC, _CC), 1)
    tri = (cc <= rc).astype(_F32)
    carry = jnp.zeros((1, _DH), _F32)
    chunks = []
    for c in range(_L // _CC):
        vc = v[c * _CC:(c + 1) * _CC, :]
        chunks.append(_mmb(tri, vc, 1, 0) + carry)
        carry = carry + jnp.sum(vc, axis=0, keepdims=True)
    ctx = jnp.concatenate(chunks, axis=0)

    selcol = _mmb(p, jnp.ones((_UPAD, 1), _F32), 0, 0)
    scat = _mmb(p, upd, 0, 0)
    o_ref[0] = ctx * (1.0 - selcol) + scat


def _proj_kernel(c_ref, wo_ref, bo_ref, o_ref):
    acc = bo_ref[...]
    for h in range(_H):
        acc = acc + _mmb(c_ref[h], wo_ref[:, h * _DH:(h + 1) * _DH], 1, 1)
    o_ref[...] = acc


def kernel(x, Wq, bq, Wk, bk, Wv, bv, Wo, bo):
    ct = jnp.asarray(_CT_NP)
    bm = jnp.asarray(_BM_NP)
    xs = x[0]
    bq2 = bq.reshape(_H, 1, _DH)
    bqc = bq.reshape(_H, _DH, 1)
    bk2 = bk.reshape(_H, 1, _DH)
    bv2 = bv.reshape(_H, 1, _DH)
    bo2 = bo.reshape(1, _D)

    ctx = pl.pallas_call(
        _head_kernel,
        grid=(_H,),
        in_specs=[
            pl.BlockSpec((_L, _D), lambda h: (0, 0)),
            pl.BlockSpec((_L, _L), lambda h: (0, 0)),
            pl.BlockSpec((_L, _L), lambda h: (0, 0)),
            pl.BlockSpec((_DH, _D), lambda h: (h, 0)),
            pl.BlockSpec((1, 1, _DH), lambda h: (h, 0, 0)),
            pl.BlockSpec((1, _DH, 1), lambda h: (h, 0, 0)),
            pl.BlockSpec((_DH, _D), lambda h: (h, 0)),
            pl.BlockSpec((1, 1, _DH), lambda h: (h, 0, 0)),
            pl.BlockSpec((_DH, _D), lambda h: (h, 0)),
            pl.BlockSpec((1, 1, _DH), lambda h: (h, 0, 0)),
        ],
        out_specs=pl.BlockSpec((1, _L, _DH), lambda h: (h, 0, 0)),
        out_shape=jax.ShapeDtypeStruct((_H, _L, _DH), _F32),
        scratch_shapes=[pltpu.VMEM((_UPAD, _L), _F32)],
    )(xs, ct, bm, Wq, bq2, bqc, Wk, bk2, Wv, bv2)

    out = pl.pallas_call(
        _proj_kernel,
        grid=(8,),
        in_specs=[
            pl.BlockSpec((_H, _L // 8, _DH), lambda i: (0, i, 0)),
            pl.BlockSpec((_D, _D), lambda i: (0, 0)),
            pl.BlockSpec((1, _D), lambda i: (0, 0)),
        ],
        out_specs=pl.BlockSpec((_L // 8, _D), lambda i: (i, 0)),
        out_shape=jax.ShapeDtypeStruct((_L, _D), _F32),
    )(ctx, Wo, bo2)
    return out[None]


# speedup vs baseline: 8.4515x; 1.3418x over previous
import math

import numpy as np
import jax
import jax.numpy as jnp
from jax.experimental import pallas as pl
from jax.experimental.pallas import tpu as pltpu

_L = 2048
_D = 768
_H = 12
_DH = 64
_U = 40
_UPAD = 64
_KC = 512
_CC = 256
_SCALE = 1.0 / math.sqrt(_DH)
_F32 = jnp.float32


def _rotl(x, r):
    return ((x << np.uint32(r)) | (x >> np.uint32(32 - r))).astype(np.uint32)


def _tf2x32_pair(k0, k1, x0, x1):
    ks = [np.uint32(k0), np.uint32(k1),
          np.uint32(np.uint32(k0) ^ np.uint32(k1) ^ np.uint32(0x1BD11BDA))]
    rot = [[13, 15, 26, 6], [17, 29, 16, 24]]
    x = [x0.astype(np.uint32) + ks[0], x1.astype(np.uint32) + ks[1]]
    for i in range(5):
        for r in rot[i % 2]:
            x[0] = x[0] + x[1]
            x[1] = _rotl(x[1], r)
            x[1] = x[1] ^ x[0]
        x[0] = x[0] + ks[(i + 1) % 3]
        x[1] = x[1] + ks[(i + 2) % 3] + np.uint32(i + 1)
    return x[0], x[1]


def _random_bits32(k0, k1, n):
    idx = np.arange(n, dtype=np.uint64)
    b1, b2 = _tf2x32_pair(k0, k1, (idx >> np.uint64(32)).astype(np.uint32),
                          (idx & np.uint64(0xFFFFFFFF)).astype(np.uint32))
    return b1 ^ b2


def _np_randint(seed, n, span):
    k0, k1 = np.uint32((seed >> 32) & 0xFFFFFFFF), np.uint32(seed & 0xFFFFFFFF)
    b1, b2 = _tf2x32_pair(k0, k1, np.zeros(2, np.uint32),
                          np.arange(2, dtype=np.uint32))
    hi = _random_bits32(b1[0], b2[0], n)
    lo = _random_bits32(b1[1], b2[1], n)
    span = np.uint32(span)
    mult = np.uint32((np.uint64(65536) % span) ** 2 % span)
    return (((hi % span) * mult + (lo % span)) % span).astype(np.int32)


def _sample_counts_t():
    idx = _np_randint(42, _L * _U, _L).reshape(_L, _U)
    c = np.zeros((_L, _L), np.float32)
    np.add.at(c, (np.arange(_L)[:, None], idx), 1.0)
    return np.ascontiguousarray(c.T)


_CT_NP = _sample_counts_t()
_BM_NP = np.where(_CT_NP > 0, np.float32(0), np.float32(-np.inf))


def _mmb(a, b, ca, cb):
    return jax.lax.dot_general(
        a.astype(jnp.bfloat16), b.astype(jnp.bfloat16),
        (((ca,), (cb,)), ((), ())), preferred_element_type=_F32)


def _mmh(a, b, ca, cb):
    return jax.lax.dot_general(
        a, b, (((ca,), (cb,)), ((), ())),
        precision=jax.lax.Precision.HIGHEST, preferred_element_type=_F32)


def _bitonic_stage(val, idx, k, d, jg):
    if d >= 128:
        s = d // 128
        pva, pvb = jnp.roll(val, -s, axis=0), jnp.roll(val, s, axis=0)
        pia, pib = jnp.roll(idx, -s, axis=0), jnp.roll(idx, s, axis=0)
    else:
        pva, pvb = jnp.roll(val, -d, axis=1), jnp.roll(val, d, axis=1)
        pia, pib = jnp.roll(idx, -d, axis=1), jnp.roll(idx, d, axis=1)
    islow = (jg & d) == 0
    pv = jnp.where(islow, pva, pvb)
    pi = jnp.where(islow, pia, pib)
    takemax = ((jg & k) == 0) == islow
    gt = (pv > val) | ((pv == val) & (pi < idx))
    takep = takemax == gt
    return jnp.where(takep, pv, val), jnp.where(takep, pi, idx)


def _head_kernel(x_ref, ct_ref, bm_ref, wq_ref, bq_ref, bqc_ref, wk_ref,
                 bk_ref, wv_ref, bv_ref, o_ref):
    x = x_ref[...]
    q = _mmb(x, wq_ref[...], 1, 1) + bq_ref[0]
    k = _mmb(x, wk_ref[...], 1, 1) + bk_ref[0]
    v = _mmb(x, wv_ref[...], 1, 1) + bv_ref[0]
    qt = _mmb(wq_ref[...], x, 1, 1) + bqc_ref[0]

    kct = _mmb(k, ct_ref[...], 0, 0)
    msum = jnp.sum(qt.astype(jnp.bfloat16).astype(_F32) * kct,
                   axis=0, keepdims=True)
    mmax = jnp.full((1, _L), -jnp.inf, _F32)
    for c in range(_L // _KC):
        kc = k[c * _KC:(c + 1) * _KC, :]
        st = _mmb(kc, q, 1, 1)
        st = st + bm_ref[c * _KC:(c + 1) * _KC, :]
        mmax = jnp.maximum(mmax, jnp.max(st, axis=0, keepdims=True))
    m = mmax - msum * (1.0 / _L)

    val = jnp.reshape(m, (16, 128))
    jg = (jax.lax.broadcasted_iota(jnp.int32, (16, 128), 0) * 128
          + jax.lax.broadcasted_iota(jnp.int32, (16, 128), 1))
    idx = jg
    k2 = 2
    while k2 <= _L:
        d = k2 // 2
        while d >= 1:
            val, idx = _bitonic_stage(val, idx, k2, d, jg)
            d //= 2
        k2 *= 2
    irow = idx[0:1, :].astype(_F32)
    e64 = (jax.lax.broadcasted_iota(jnp.int32, (_UPAD, 128), 0)
           == jax.lax.broadcasted_iota(jnp.int32, (_UPAD, 128), 1)).astype(_F32)
    idxcol = _mmh(e64, irow, 1, 1)
    urow = jax.lax.broadcasted_iota(jnp.int32, (_UPAD, 1), 0)
    valid = urow < _U
    pos_sel = jnp.where(valid, idxcol, -1.0)
    qpos = jnp.where(valid, idxcol, 0.0)
    iot = jax.lax.broadcasted_iota(jnp.int32, (1, _L), 1)
    iotf = iot.astype(_F32)
    p = (pos_sel == iotf).astype(_F32)

    qs = _mmb(p, q, 1, 0)
    sc = _mmb(qs, k, 1, 1) * _SCALE
    sc = jnp.where(iotf > qpos, -jnp.inf, sc)
    smax = jnp.max(sc, axis=1, keepdims=True)
    e = jnp.exp(sc - smax)
    attn = e / jnp.sum(e, axis=1, keepdims=True)
    upd = _mmb(attn, v, 1, 0)

    rc = jax.lax.broadcasted_iota(jnp.int32, (_CC, _CC), 0)
    cc = jax.lax.broadcasted_iota(jnp.int32, (_CC, _CC), 1)
    tri = (cc <= rc).astype(_F32)
    carry = jnp.zeros((1, _DH), _F32)
    chunks = []
    for c in range(_L // _CC):
        vc = v[c * _CC:(c + 1) * _CC, :]
        chunks.append(_mmb(tri, vc, 1, 0) + carry)
        carry = carry + jnp.sum(vc, axis=0, keepdims=True)
    ctx = jnp.concatenate(chunks, axis=0)

    selcol = _mmb(p, jnp.ones((_UPAD, 1), _F32), 0, 0)
    scat = _mmb(p, upd, 0, 0)
    o_ref[0] = ctx * (1.0 - selcol) + scat


def _proj_kernel(c_ref, wo_ref, bo_ref, o_ref):
    acc = bo_ref[...]
    for h in range(_H):
        acc = acc + _mmb(c_ref[h], wo_ref[:, h * _DH:(h + 1) * _DH], 1, 1)
    o_ref[...] = acc


def kernel(x, Wq, bq, Wk, bk, Wv, bv, Wo, bo):
    ct = jnp.asarray(_CT_NP)
    bm = jnp.asarray(_BM_NP)
    xs = x[0]
    bq2 = bq.reshape(_H, 1, _DH)
    bqc = bq.reshape(_H, _DH, 1)
    bk2 = bk.reshape(_H, 1, _DH)
    bv2 = bv.reshape(_H, 1, _DH)
    bo2 = bo.reshape(1, _D)

    ctx = pl.pallas_call(
        _head_kernel,
        grid=(_H,),
        in_specs=[
            pl.BlockSpec((_L, _D), lambda h: (0, 0)),
            pl.BlockSpec((_L, _L), lambda h: (0, 0)),
            pl.BlockSpec((_L, _L), lambda h: (0, 0)),
            pl.BlockSpec((_DH, _D), lambda h: (h, 0)),
            pl.BlockSpec((1, 1, _DH), lambda h: (h, 0, 0)),
            pl.BlockSpec((1, _DH, 1), lambda h: (h, 0, 0)),
            pl.BlockSpec((_DH, _D), lambda h: (h, 0)),
            pl.BlockSpec((1, 1, _DH), lambda h: (h, 0, 0)),
            pl.BlockSpec((_DH, _D), lambda h: (h, 0)),
            pl.BlockSpec((1, 1, _DH), lambda h: (h, 0, 0)),
        ],
        out_specs=pl.BlockSpec((1, _L, _DH), lambda h: (h, 0, 0)),
        out_shape=jax.ShapeDtypeStruct((_H, _L, _DH), _F32),
    )(xs, ct, bm, Wq, bq2, bqc, Wk, bk2, Wv, bv2)

    out = pl.pallas_call(
        _proj_kernel,
        grid=(8,),
        in_specs=[
            pl.BlockSpec((_H, _L // 8, _DH), lambda i: (0, i, 0)),
            pl.BlockSpec((_D, _D), lambda i: (0, 0)),
            pl.BlockSpec((1, _D), lambda i: (0, 0)),
        ],
        out_specs=pl.BlockSpec((_L // 8, _D), lambda i: (i, 0)),
        out_shape=jax.ShapeDtypeStruct((_L, _D), _F32),
    )(ctx, Wo, bo2)
    return out[None]


# speedup vs baseline: 8.7237x; 1.0322x over previous
import math

import numpy as np
import jax
import jax.numpy as jnp
from jax.experimental import pallas as pl
from jax.experimental.pallas import tpu as pltpu

_L = 2048
_D = 768
_H = 12
_DH = 64
_U = 40
_UPAD = 64
_KC = 512
_HPB = 2
_CC = 256
_SCALE = 1.0 / math.sqrt(_DH)
_F32 = jnp.float32


def _rotl(x, r):
    return ((x << np.uint32(r)) | (x >> np.uint32(32 - r))).astype(np.uint32)


def _tf2x32_pair(k0, k1, x0, x1):
    ks = [np.uint32(k0), np.uint32(k1),
          np.uint32(np.uint32(k0) ^ np.uint32(k1) ^ np.uint32(0x1BD11BDA))]
    rot = [[13, 15, 26, 6], [17, 29, 16, 24]]
    x = [x0.astype(np.uint32) + ks[0], x1.astype(np.uint32) + ks[1]]
    for i in range(5):
        for r in rot[i % 2]:
            x[0] = x[0] + x[1]
            x[1] = _rotl(x[1], r)
            x[1] = x[1] ^ x[0]
        x[0] = x[0] + ks[(i + 1) % 3]
        x[1] = x[1] + ks[(i + 2) % 3] + np.uint32(i + 1)
    return x[0], x[1]


def _random_bits32(k0, k1, n):
    idx = np.arange(n, dtype=np.uint64)
    b1, b2 = _tf2x32_pair(k0, k1, (idx >> np.uint64(32)).astype(np.uint32),
                          (idx & np.uint64(0xFFFFFFFF)).astype(np.uint32))
    return b1 ^ b2


def _np_randint(seed, n, span):
    k0, k1 = np.uint32((seed >> 32) & 0xFFFFFFFF), np.uint32(seed & 0xFFFFFFFF)
    b1, b2 = _tf2x32_pair(k0, k1, np.zeros(2, np.uint32),
                          np.arange(2, dtype=np.uint32))
    hi = _random_bits32(b1[0], b2[0], n)
    lo = _random_bits32(b1[1], b2[1], n)
    span = np.uint32(span)
    mult = np.uint32((np.uint64(65536) % span) ** 2 % span)
    return (((hi % span) * mult + (lo % span)) % span).astype(np.int32)


def _sample_counts_t():
    idx = _np_randint(42, _L * _U, _L).reshape(_L, _U)
    c = np.zeros((_L, _L), np.float32)
    np.add.at(c, (np.arange(_L)[:, None], idx), 1.0)
    return np.ascontiguousarray(c.T)


_CT_NP = _sample_counts_t()
_BM_NP = np.where(_CT_NP > 0, np.float32(0), np.float32(-np.inf))


def _mmb(a, b, ca, cb):
    return jax.lax.dot_general(
        a.astype(jnp.bfloat16), b.astype(jnp.bfloat16),
        (((ca,), (cb,)), ((), ())), preferred_element_type=_F32)


def _mmh(a, b, ca, cb):
    return jax.lax.dot_general(
        a, b, (((ca,), (cb,)), ((), ())),
        precision=jax.lax.Precision.HIGHEST, preferred_element_type=_F32)


def _bitonic_stage(val, idx, k, d, jg):
    if d >= 128:
        s = d // 128
        pva, pvb = jnp.roll(val, -s, axis=0), jnp.roll(val, s, axis=0)
        pia, pib = jnp.roll(idx, -s, axis=0), jnp.roll(idx, s, axis=0)
    else:
        pva, pvb = jnp.roll(val, -d, axis=1), jnp.roll(val, d, axis=1)
        pia, pib = jnp.roll(idx, -d, axis=1), jnp.roll(idx, d, axis=1)
    islow = (jg & d) == 0
    pv = jnp.where(islow, pva, pvb)
    pi = jnp.where(islow, pia, pib)
    takemax = ((jg & k) == 0) == islow
    gt = (pv > val) | ((pv == val) & (pi < idx))
    takep = takemax == gt
    return jnp.where(takep, pv, val), jnp.where(takep, pi, idx)


def _one_head(x, ct, bm, wq, bqr, bqc, wk, bkr, wv, bvr):
    q = _mmb(x, wq, 1, 1) + bqr
    k = _mmb(x, wk, 1, 1) + bkr
    v = _mmb(x, wv, 1, 1) + bvr
    qt = _mmb(wq, x, 1, 1) + bqc

    kct = _mmb(k, ct, 0, 0)
    msum = jnp.sum(qt.astype(jnp.bfloat16).astype(_F32) * kct,
                   axis=0, keepdims=True)
    mmax = jnp.full((1, _L), -jnp.inf, _F32)
    for c in range(_L // _KC):
        kc = k[c * _KC:(c + 1) * _KC, :]
        st = _mmb(kc, q, 1, 1)
        st = st + bm[c * _KC:(c + 1) * _KC, :]
        mmax = jnp.maximum(mmax, jnp.max(st, axis=0, keepdims=True))
    m = mmax - msum * (1.0 / _L)

    val = jnp.reshape(m, (16, 128))
    jg = (jax.lax.broadcasted_iota(jnp.int32, (16, 128), 0) * 128
          + jax.lax.broadcasted_iota(jnp.int32, (16, 128), 1))
    idx = jg
    k2 = 2
    while k2 <= _L:
        d = k2 // 2
        while d >= 1:
            val, idx = _bitonic_stage(val, idx, k2, d, jg)
            d //= 2
        k2 *= 2
    irow = idx[0:1, :].astype(_F32)
    e64 = (jax.lax.broadcasted_iota(jnp.int32, (_UPAD, 128), 0)
           == jax.lax.broadcasted_iota(jnp.int32, (_UPAD, 128), 1)).astype(_F32)
    idxcol = _mmh(e64, irow, 1, 1)
    urow = jax.lax.broadcasted_iota(jnp.int32, (_UPAD, 1), 0)
    valid = urow < _U
    pos_sel = jnp.where(valid, idxcol, -1.0)
    qpos = jnp.where(valid, idxcol, 0.0)
    iot = jax.lax.broadcasted_iota(jnp.int32, (1, _L), 1)
    iotf = iot.astype(_F32)
    p = (pos_sel == iotf).astype(_F32)

    qs = _mmb(p, q, 1, 0)
    sc = _mmb(qs, k, 1, 1) * _SCALE
    sc = jnp.where(iotf > qpos, -jnp.inf, sc)
    smax = jnp.max(sc, axis=1, keepdims=True)
    e = jnp.exp(sc - smax)
    attn = e / jnp.sum(e, axis=1, keepdims=True)
    upd = _mmb(attn, v, 1, 0)

    rc = jax.lax.broadcasted_iota(jnp.int32, (_CC, _CC), 0)
    cc = jax.lax.broadcasted_iota(jnp.int32, (_CC, _CC), 1)
    tri = (cc <= rc).astype(_F32)
    carry = jnp.zeros((1, _DH), _F32)
    chunks = []
    for c in range(_L // _CC):
        vc = v[c * _CC:(c + 1) * _CC, :]
        chunks.append(_mmb(tri, vc, 1, 0) + carry)
        carry = carry + jnp.sum(vc, axis=0, keepdims=True)
    ctx = jnp.concatenate(chunks, axis=0)

    selcol = _mmb(p, jnp.ones((_UPAD, 1), _F32), 0, 0)
    scat = _mmb(p, upd, 0, 0)
    return ctx * (1.0 - selcol) + scat


def _head_kernel(x_ref, ct_ref, bm_ref, wq_ref, bq_ref, bqc_ref, wk_ref,
                 bk_ref, wv_ref, bv_ref, o_ref):
    x = x_ref[...]
    ct = ct_ref[...]
    bm = bm_ref[...]
    for hh in range(_HPB):
        sl = slice(hh * _DH, (hh + 1) * _DH)
        o_ref[hh] = _one_head(x, ct, bm, wq_ref[sl, :], bq_ref[hh],
                              bqc_ref[hh], wk_ref[sl, :], bk_ref[hh],
                              wv_ref[sl, :], bv_ref[hh])


def _proj_kernel(c_ref, wo_ref, bo_ref, o_ref):
    acc = bo_ref[...]
    for h in range(_H):
        acc = acc + _mmb(c_ref[h], wo_ref[:, h * _DH:(h + 1) * _DH], 1, 1)
    o_ref[...] = acc


def kernel(x, Wq, bq, Wk, bk, Wv, bv, Wo, bo):
    ct = jnp.asarray(_CT_NP).astype(jnp.bfloat16)
    bm = jnp.asarray(_BM_NP).astype(jnp.bfloat16)
    xs = x[0]
    bq2 = bq.reshape(_H, 1, _DH)
    bqc = bq.reshape(_H, _DH, 1)
    bk2 = bk.reshape(_H, 1, _DH)
    bv2 = bv.reshape(_H, 1, _DH)
    bo2 = bo.reshape(1, _D)

    ctx = pl.pallas_call(
        _head_kernel,
        grid=(_H // _HPB,),
        in_specs=[
            pl.BlockSpec((_L, _D), lambda h: (0, 0)),
            pl.BlockSpec((_L, _L), lambda h: (0, 0)),
            pl.BlockSpec((_L, _L), lambda h: (0, 0)),
            pl.BlockSpec((_HPB * _DH, _D), lambda h: (h, 0)),
            pl.BlockSpec((_HPB, 1, _DH), lambda h: (h, 0, 0)),
            pl.BlockSpec((_HPB, _DH, 1), lambda h: (h, 0, 0)),
            pl.BlockSpec((_HPB * _DH, _D), lambda h: (h, 0)),
            pl.BlockSpec((_HPB, 1, _DH), lambda h: (h, 0, 0)),
            pl.BlockSpec((_HPB * _DH, _D), lambda h: (h, 0)),
            pl.BlockSpec((_HPB, 1, _DH), lambda h: (h, 0, 0)),
        ],
        out_specs=pl.BlockSpec((_HPB, _L, _DH), lambda h: (h, 0, 0)),
        out_shape=jax.ShapeDtypeStruct((_H, _L, _DH), _F32),
    )(xs, ct, bm, Wq, bq2, bqc, Wk, bk2, Wv, bv2)

    out = pl.pallas_call(
        _proj_kernel,
        grid=(8,),
        in_specs=[
            pl.BlockSpec((_H, _L // 8, _DH), lambda i: (0, i, 0)),
            pl.BlockSpec((_D, _D), lambda i: (0, 0)),
            pl.BlockSpec((1, _D), lambda i: (0, 0)),
        ],
        out_specs=pl.BlockSpec((_L // 8, _D), lambda i: (i, 0)),
        out_shape=jax.ShapeDtypeStruct((_L, _D), _F32),
    )(ctx, Wo, bo2)
    return out[None]


# speedup vs baseline: 10.1426x; 1.1626x over previous
import math

import numpy as np
import jax
import jax.numpy as jnp
from jax.experimental import pallas as pl
from jax.experimental.pallas import tpu as pltpu

_L = 2048
_D = 768
_H = 12
_DH = 64
_U = 40
_UPAD = 64
_KC = 512
_HPB = 2
_CC = 256
_SCALE = 1.0 / math.sqrt(_DH)
_F32 = jnp.float32


def _rotl(x, r):
    return ((x << np.uint32(r)) | (x >> np.uint32(32 - r))).astype(np.uint32)


def _tf2x32_pair(k0, k1, x0, x1):
    ks = [np.uint32(k0), np.uint32(k1),
          np.uint32(np.uint32(k0) ^ np.uint32(k1) ^ np.uint32(0x1BD11BDA))]
    rot = [[13, 15, 26, 6], [17, 29, 16, 24]]
    x = [x0.astype(np.uint32) + ks[0], x1.astype(np.uint32) + ks[1]]
    for i in range(5):
        for r in rot[i % 2]:
            x[0] = x[0] + x[1]
            x[1] = _rotl(x[1], r)
            x[1] = x[1] ^ x[0]
        x[0] = x[0] + ks[(i + 1) % 3]
        x[1] = x[1] + ks[(i + 2) % 3] + np.uint32(i + 1)
    return x[0], x[1]


def _random_bits32(k0, k1, n):
    idx = np.arange(n, dtype=np.uint64)
    b1, b2 = _tf2x32_pair(k0, k1, (idx >> np.uint64(32)).astype(np.uint32),
                          (idx & np.uint64(0xFFFFFFFF)).astype(np.uint32))
    return b1 ^ b2


def _np_randint(seed, n, span):
    k0, k1 = np.uint32((seed >> 32) & 0xFFFFFFFF), np.uint32(seed & 0xFFFFFFFF)
    b1, b2 = _tf2x32_pair(k0, k1, np.zeros(2, np.uint32),
                          np.arange(2, dtype=np.uint32))
    hi = _random_bits32(b1[0], b2[0], n)
    lo = _random_bits32(b1[1], b2[1], n)
    span = np.uint32(span)
    mult = np.uint32((np.uint64(65536) % span) ** 2 % span)
    return (((hi % span) * mult + (lo % span)) % span).astype(np.int32)


def _sample_counts_t():
    idx = _np_randint(42, _L * _U, _L).reshape(_L, _U)
    c = np.zeros((_L, _L), np.float32)
    np.add.at(c, (np.arange(_L)[:, None], idx), 1.0)
    return np.ascontiguousarray(c.T)


_CT_NP = _sample_counts_t()
_BM_NP = np.where(_CT_NP > 0, np.float32(0), np.float32(-np.inf))


def _mmb(a, b, ca, cb):
    return jax.lax.dot_general(
        a.astype(jnp.bfloat16), b.astype(jnp.bfloat16),
        (((ca,), (cb,)), ((), ())), preferred_element_type=_F32)


def _mmh(a, b, ca, cb):
    return jax.lax.dot_general(
        a, b, (((ca,), (cb,)), ((), ())),
        precision=jax.lax.Precision.HIGHEST, preferred_element_type=_F32)


def _bitonic_stage(val, idx, k, d, jg):
    if d >= 128:
        s = d // 128
        pva, pvb = jnp.roll(val, -s, axis=0), jnp.roll(val, s, axis=0)
        pia, pib = jnp.roll(idx, -s, axis=0), jnp.roll(idx, s, axis=0)
    else:
        pva, pvb = jnp.roll(val, -d, axis=1), jnp.roll(val, d, axis=1)
        pia, pib = jnp.roll(idx, -d, axis=1), jnp.roll(idx, d, axis=1)
    islow = (jg & d) == 0
    pv = jnp.where(islow, pva, pvb)
    pi = jnp.where(islow, pia, pib)
    takemax = ((jg & k) == 0) == islow
    gt = (pv > val) | ((pv == val) & (pi < idx))
    takep = takemax == gt
    return jnp.where(takep, pv, val), jnp.where(takep, pi, idx)


def _head_pre(x, ct, bm, wq, bqr, bqc, wk, bkr, wv, bvr):
    q = _mmb(x, wq, 1, 1) + bqr
    k = _mmb(x, wk, 1, 1) + bkr
    v = _mmb(x, wv, 1, 1) + bvr
    qt = _mmb(wq, x, 1, 1) + bqc

    kct = _mmb(k, ct, 0, 0)
    msum = jnp.sum(qt.astype(jnp.bfloat16).astype(_F32) * kct,
                   axis=0, keepdims=True)
    mmax = jnp.full((1, _L), -jnp.inf, _F32)
    for c in range(_L // _KC):
        kc = k[c * _KC:(c + 1) * _KC, :]
        st = _mmb(kc, q, 1, 1)
        st = st + bm[c * _KC:(c + 1) * _KC, :]
        mmax = jnp.maximum(mmax, jnp.max(st, axis=0, keepdims=True))
    m = mmax - msum * (1.0 / _L)
    return q, k, v, m


def _fused_topk(ms):
    nh = len(ms)
    val = jnp.concatenate([jnp.reshape(m, (16, 128)) for m in ms], axis=0)
    row = jax.lax.broadcasted_iota(jnp.int32, (16 * nh, 128), 0)
    lane = jax.lax.broadcasted_iota(jnp.int32, (16 * nh, 128), 1)
    jg = (row & 15) * 128 + lane
    idx = jg
    k2 = 2
    while k2 <= _L:
        d = k2 // 2
        while d >= 1:
            val, idx = _bitonic_stage(val, idx, k2, d, jg)
            d //= 2
        k2 *= 2
    return [idx[16 * h:16 * h + 1, :].astype(_F32) for h in range(nh)]


def _head_post(q, k, v, irow, x, o_ref, hh):
    e64 = (jax.lax.broadcasted_iota(jnp.int32, (_UPAD, 128), 0)
           == jax.lax.broadcasted_iota(jnp.int32, (_UPAD, 128), 1)).astype(_F32)
    idxcol = _mmh(e64, irow, 1, 1)
    urow = jax.lax.broadcasted_iota(jnp.int32, (_UPAD, 1), 0)
    valid = urow < _U
    pos_sel = jnp.where(valid, idxcol, -1.0)
    qpos = jnp.where(valid, idxcol, 0.0)
    iot = jax.lax.broadcasted_iota(jnp.int32, (1, _L), 1)
    iotf = iot.astype(_F32)
    p = (pos_sel == iotf).astype(_F32)

    qs = _mmb(p, q, 1, 0)
    sc = _mmb(qs, k, 1, 1) * _SCALE
    sc = jnp.where(iotf > qpos, -jnp.inf, sc)
    smax = jnp.max(sc, axis=1, keepdims=True)
    e = jnp.exp(sc - smax)
    attn = e / jnp.sum(e, axis=1, keepdims=True)
    upd = _mmb(attn, v, 1, 0)

    rc = jax.lax.broadcasted_iota(jnp.int32, (_CC, _CC), 0)
    cc = jax.lax.broadcasted_iota(jnp.int32, (_CC, _CC), 1)
    tri = (cc <= rc).astype(_F32)
    carry = jnp.zeros((1, _DH), _F32)
    chunks = []
    for c in range(_L // _CC):
        vc = v[c * _CC:(c + 1) * _CC, :]
        chunks.append(_mmb(tri, vc, 1, 0) + carry)
        carry = carry + jnp.sum(vc, axis=0, keepdims=True)
    ctx = jnp.concatenate(chunks, axis=0)

    selcol = _mmb(p, jnp.ones((_UPAD, 1), _F32), 0, 0)
    scat = _mmb(p, upd, 0, 0)
    o_ref[hh] = ctx * (1.0 - selcol) + scat


def _head_kernel(x_ref, ct_ref, bm_ref, wq_ref, bq_ref, bqc_ref, wk_ref,
                 bk_ref, wv_ref, bv_ref, o_ref):
    x = x_ref[...]
    ct = ct_ref[...]
    bm = bm_ref[...]
    pre = []
    for hh in range(_HPB):
        sl = slice(hh * _DH, (hh + 1) * _DH)
        pre.append(_head_pre(x, ct, bm, wq_ref[sl, :], bq_ref[hh],
                             bqc_ref[hh], wk_ref[sl, :], bk_ref[hh],
                             wv_ref[sl, :], bv_ref[hh]))
    irows = _fused_topk([t[3] for t in pre])
    for hh in range(_HPB):
        q, k, v, _ = pre[hh]
        _head_post(q, k, v, irows[hh], x, o_ref, hh)


def _proj_kernel(c_ref, wo_ref, bo_ref, o_ref):
    acc = bo_ref[...]
    for h in range(_H):
        acc = acc + _mmb(c_ref[h], wo_ref[:, h * _DH:(h + 1) * _DH], 1, 1)
    o_ref[...] = acc


def kernel(x, Wq, bq, Wk, bk, Wv, bv, Wo, bo):
    ct = jnp.asarray(_CT_NP).astype(jnp.bfloat16)
    bm = jnp.asarray(_BM_NP).astype(jnp.bfloat16)
    xs = x[0]
    bq2 = bq.reshape(_H, 1, _DH)
    bqc = bq.reshape(_H, _DH, 1)
    bk2 = bk.reshape(_H, 1, _DH)
    bv2 = bv.reshape(_H, 1, _DH)
    bo2 = bo.reshape(1, _D)

    ctx = pl.pallas_call(
        _head_kernel,
        grid=(_H // _HPB,),
        in_specs=[
            pl.BlockSpec((_L, _D), lambda h: (0, 0)),
            pl.BlockSpec((_L, _L), lambda h: (0, 0)),
            pl.BlockSpec((_L, _L), lambda h: (0, 0)),
            pl.BlockSpec((_HPB * _DH, _D), lambda h: (h, 0)),
            pl.BlockSpec((_HPB, 1, _DH), lambda h: (h, 0, 0)),
            pl.BlockSpec((_HPB, _DH, 1), lambda h: (h, 0, 0)),
            pl.BlockSpec((_HPB * _DH, _D), lambda h: (h, 0)),
            pl.BlockSpec((_HPB, 1, _DH), lambda h: (h, 0, 0)),
            pl.BlockSpec((_HPB * _DH, _D), lambda h: (h, 0)),
            pl.BlockSpec((_HPB, 1, _DH), lambda h: (h, 0, 0)),
        ],
        out_specs=pl.BlockSpec((_HPB, _L, _DH), lambda h: (h, 0, 0)),
        out_shape=jax.ShapeDtypeStruct((_H, _L, _DH), _F32),
    )(xs, ct, bm, Wq, bq2, bqc, Wk, bk2, Wv, bv2)

    out = pl.pallas_call(
        _proj_kernel,
        grid=(8,),
        in_specs=[
            pl.BlockSpec((_H, _L // 8, _DH), lambda i: (0, i, 0)),
            pl.BlockSpec((_D, _D), lambda i: (0, 0)),
            pl.BlockSpec((1, _D), lambda i: (0, 0)),
        ],
        out_specs=pl.BlockSpec((_L // 8, _D), lambda i: (i, 0)),
        out_shape=jax.ShapeDtypeStruct((_L, _D), _F32),
    )(ctx, Wo, bo2)
    return out[None]


# speedup vs baseline: 10.3039x; 1.0159x over previous
import math

import numpy as np
import jax
import jax.numpy as jnp
from jax.experimental import pallas as pl
from jax.experimental.pallas import tpu as pltpu

_L = 2048
_D = 768
_H = 12
_DH = 64
_U = 40
_UPAD = 64
_KC = 512
_HPB = 2
_CC = 256
_SCALE = 1.0 / math.sqrt(_DH)
_F32 = jnp.float32


def _rotl(x, r):
    return ((x << np.uint32(r)) | (x >> np.uint32(32 - r))).astype(np.uint32)


def _tf2x32_pair(k0, k1, x0, x1):
    ks = [np.uint32(k0), np.uint32(k1),
          np.uint32(np.uint32(k0) ^ np.uint32(k1) ^ np.uint32(0x1BD11BDA))]
    rot = [[13, 15, 26, 6], [17, 29, 16, 24]]
    x = [x0.astype(np.uint32) + ks[0], x1.astype(np.uint32) + ks[1]]
    for i in range(5):
        for r in rot[i % 2]:
            x[0] = x[0] + x[1]
            x[1] = _rotl(x[1], r)
            x[1] = x[1] ^ x[0]
        x[0] = x[0] + ks[(i + 1) % 3]
        x[1] = x[1] + ks[(i + 2) % 3] + np.uint32(i + 1)
    return x[0], x[1]


def _random_bits32(k0, k1, n):
    idx = np.arange(n, dtype=np.uint64)
    b1, b2 = _tf2x32_pair(k0, k1, (idx >> np.uint64(32)).astype(np.uint32),
                          (idx & np.uint64(0xFFFFFFFF)).astype(np.uint32))
    return b1 ^ b2


def _np_randint(seed, n, span):
    k0, k1 = np.uint32((seed >> 32) & 0xFFFFFFFF), np.uint32(seed & 0xFFFFFFFF)
    b1, b2 = _tf2x32_pair(k0, k1, np.zeros(2, np.uint32),
                          np.arange(2, dtype=np.uint32))
    hi = _random_bits32(b1[0], b2[0], n)
    lo = _random_bits32(b1[1], b2[1], n)
    span = np.uint32(span)
    mult = np.uint32((np.uint64(65536) % span) ** 2 % span)
    return (((hi % span) * mult + (lo % span)) % span).astype(np.int32)


def _sample_counts_t():
    idx = _np_randint(42, _L * _U, _L).reshape(_L, _U)
    c = np.zeros((_L, _L), np.float32)
    np.add.at(c, (np.arange(_L)[:, None], idx), 1.0)
    return np.ascontiguousarray(c.T)


_CT_NP = _sample_counts_t()
_BM_NP = np.where(_CT_NP > 0, np.float32(0), np.float32(-np.inf))


def _mmb(a, b, ca, cb):
    return jax.lax.dot_general(
        a.astype(jnp.bfloat16), b.astype(jnp.bfloat16),
        (((ca,), (cb,)), ((), ())), preferred_element_type=_F32)


def _mmh(a, b, ca, cb):
    return jax.lax.dot_general(
        a, b, (((ca,), (cb,)), ((), ())),
        precision=jax.lax.Precision.HIGHEST, preferred_element_type=_F32)


def _bitonic_stage(val, k, d, jg):
    if d >= 128:
        s = d // 128
        pva, pvb = jnp.roll(val, -s, axis=0), jnp.roll(val, s, axis=0)
    else:
        pva, pvb = jnp.roll(val, -d, axis=1), jnp.roll(val, d, axis=1)
    islow = (jg & d) == 0
    pv = jnp.where(islow, pva, pvb)
    takemax = ((jg & k) == 0) == islow
    takep = takemax == (pv > val)
    return jnp.where(takep, pv, val)


def _head_pre(x, ct, bm, wq, bqr, bqc, wk, bkr, wv, bvr):
    q = _mmb(x, wq, 1, 1) + bqr
    k = _mmb(x, wk, 1, 1) + bkr
    v = _mmb(x, wv, 1, 1) + bvr
    qt = _mmb(wq, x, 1, 1) + bqc

    kct = _mmb(k, ct, 0, 0)
    msum = jnp.sum(qt.astype(jnp.bfloat16).astype(_F32) * kct,
                   axis=0, keepdims=True)
    mmax = jnp.full((1, _L), -jnp.inf, _F32)
    for c in range(_L // _KC):
        kc = k[c * _KC:(c + 1) * _KC, :]
        st = _mmb(kc, q, 1, 1)
        st = st + bm[c * _KC:(c + 1) * _KC, :]
        mmax = jnp.maximum(mmax, jnp.max(st, axis=0, keepdims=True))
    m = mmax - msum * (1.0 / _L)
    return q, k, v, m


def _fused_topk(ms):
    nh = len(ms)
    iot = jax.lax.broadcasted_iota(jnp.int32, (1, _L), 1)
    keys = []
    for m in ms:
        ki = (jax.lax.bitcast_convert_type(m, jnp.int32) & ~2047) | (2047 - iot)
        keys.append(jnp.reshape(jax.lax.bitcast_convert_type(ki, _F32),
                                (16, 128)))
    val = jnp.concatenate(keys, axis=0)
    row = jax.lax.broadcasted_iota(jnp.int32, (16 * nh, 128), 0)
    lane = jax.lax.broadcasted_iota(jnp.int32, (16 * nh, 128), 1)
    jg = (row & 15) * 128 + lane
    k2 = 2
    while k2 <= _L:
        d = k2 // 2
        while d >= 1:
            val = _bitonic_stage(val, k2, d, jg)
            d //= 2
        k2 *= 2
    vi = jax.lax.bitcast_convert_type(val, jnp.int32)
    irows = 2047 - (vi & 2047)
    return [irows[16 * h:16 * h + 1, :].astype(_F32) for h in range(nh)]


def _head_post(q, k, v, irow, x, o_ref, hh):
    e64 = (jax.lax.broadcasted_iota(jnp.int32, (_UPAD, 128), 0)
           == jax.lax.broadcasted_iota(jnp.int32, (_UPAD, 128), 1)).astype(_F32)
    idxcol = _mmh(e64, irow, 1, 1)
    urow = jax.lax.broadcasted_iota(jnp.int32, (_UPAD, 1), 0)
    valid = urow < _U
    pos_sel = jnp.where(valid, idxcol, -1.0)
    qpos = jnp.where(valid, idxcol, 0.0)
    iot = jax.lax.broadcasted_iota(jnp.int32, (1, _L), 1)
    iotf = iot.astype(_F32)
    p = (pos_sel == iotf).astype(_F32)

    qs = _mmb(p, q, 1, 0)
    sc = _mmb(qs, k, 1, 1) * _SCALE
    sc = jnp.where(iotf > qpos, -jnp.inf, sc)
    smax = jnp.max(sc, axis=1, keepdims=True)
    e = jnp.exp(sc - smax)
    attn = e / jnp.sum(e, axis=1, keepdims=True)
    upd = _mmb(attn, v, 1, 0)

    rc = jax.lax.broadcasted_iota(jnp.int32, (_CC, _CC), 0)
    cc = jax.lax.broadcasted_iota(jnp.int32, (_CC, _CC), 1)
    tri = (cc <= rc).astype(_F32)
    carry = jnp.zeros((1, _DH), _F32)
    chunks = []
    for c in range(_L // _CC):
        vc = v[c * _CC:(c + 1) * _CC, :]
        chunks.append(_mmb(tri, vc, 1, 0) + carry)
        carry = carry + jnp.sum(vc, axis=0, keepdims=True)
    ctx = jnp.concatenate(chunks, axis=0)

    selcol = _mmb(p, jnp.ones((_UPAD, 1), _F32), 0, 0)
    scat = _mmb(p, upd, 0, 0)
    o_ref[hh] = ctx * (1.0 - selcol) + scat


def _head_kernel(x_ref, ct_ref, bm_ref, wq_ref, bq_ref, bqc_ref, wk_ref,
                 bk_ref, wv_ref, bv_ref, o_ref):
    x = x_ref[...]
    ct = ct_ref[...]
    bm = bm_ref[...]
    pre = []
    for hh in range(_HPB):
        sl = slice(hh * _DH, (hh + 1) * _DH)
        pre.append(_head_pre(x, ct, bm, wq_ref[sl, :], bq_ref[hh],
                             bqc_ref[hh], wk_ref[sl, :], bk_ref[hh],
                             wv_ref[sl, :], bv_ref[hh]))
    irows = _fused_topk([t[3] for t in pre])
    for hh in range(_HPB):
        q, k, v, _ = pre[hh]
        _head_post(q, k, v, irows[hh], x, o_ref, hh)


def _proj_kernel(c_ref, wo_ref, bo_ref, o_ref):
    acc = bo_ref[...]
    for h in range(_H):
        acc = acc + _mmb(c_ref[h], wo_ref[:, h * _DH:(h + 1) * _DH], 1, 1)
    o_ref[...] = acc


def kernel(x, Wq, bq, Wk, bk, Wv, bv, Wo, bo):
    ct = jnp.asarray(_CT_NP).astype(jnp.bfloat16)
    bm = jnp.asarray(_BM_NP).astype(jnp.bfloat16)
    xs = x[0]
    bq2 = bq.reshape(_H, 1, _DH)
    bqc = bq.reshape(_H, _DH, 1)
    bk2 = bk.reshape(_H, 1, _DH)
    bv2 = bv.reshape(_H, 1, _DH)
    bo2 = bo.reshape(1, _D)

    ctx = pl.pallas_call(
        _head_kernel,
        grid=(_H // _HPB,),
        in_specs=[
            pl.BlockSpec((_L, _D), lambda h: (0, 0)),
            pl.BlockSpec((_L, _L), lambda h: (0, 0)),
            pl.BlockSpec((_L, _L), lambda h: (0, 0)),
            pl.BlockSpec((_HPB * _DH, _D), lambda h: (h, 0)),
            pl.BlockSpec((_HPB, 1, _DH), lambda h: (h, 0, 0)),
            pl.BlockSpec((_HPB, _DH, 1), lambda h: (h, 0, 0)),
            pl.BlockSpec((_HPB * _DH, _D), lambda h: (h, 0)),
            pl.BlockSpec((_HPB, 1, _DH), lambda h: (h, 0, 0)),
            pl.BlockSpec((_HPB * _DH, _D), lambda h: (h, 0)),
            pl.BlockSpec((_HPB, 1, _DH), lambda h: (h, 0, 0)),
        ],
        out_specs=pl.BlockSpec((_HPB, _L, _DH), lambda h: (h, 0, 0)),
        out_shape=jax.ShapeDtypeStruct((_H, _L, _DH), _F32),
    )(xs, ct, bm, Wq, bq2, bqc, Wk, bk2, Wv, bv2)

    out = pl.pallas_call(
        _proj_kernel,
        grid=(8,),
        in_specs=[
            pl.BlockSpec((_H, _L // 8, _DH), lambda i: (0, i, 0)),
            pl.BlockSpec((_D, _D), lambda i: (0, 0)),
            pl.BlockSpec((1, _D), lambda i: (0, 0)),
        ],
        out_specs=pl.BlockSpec((_L // 8, _D), lambda i: (i, 0)),
        out_shape=jax.ShapeDtypeStruct((_L, _D), _F32),
    )(ctx, Wo, bo2)
    return out[None]
